# baseline (speedup 1.0000x reference)
import functools

import jax
import jax.numpy as jnp
from jax import lax
from jax.experimental import pallas as pl
from jax.experimental.pallas import tpu as pltpu
from jax.experimental.pallas import tpu_sc as plsc

VOCAB = 1000000
EMBED_DIM = 64
PAD_DIM = 128
BATCH = 4096
SEQ = 50

_INFO = plsc.get_sparse_core_info()
NC, NS = _INFO.num_cores, _INFO.num_subcores
NW = NC * NS

N_ROWS = BATCH * SEQ
PER_W = N_ROWS // NW
CHUNK = 400
N_CHUNKS = PER_W // CHUNK
NBUF = 4


def _emb_kernel(idx_hbm, table_hbm, out_hbm, idx_v, rows_v, gsems, wsems):
    wid = lax.axis_index("s") * NC + lax.axis_index("c")
    base_c = wid * N_CHUNKS
    base_r = wid * PER_W

    pltpu.sync_copy(idx_hbm.at[pl.ds(base_c, N_CHUNKS)], idx_v)

    writes = [None] * NBUF
    for step in range(N_CHUNKS):
        buf = step % NBUF
        row0 = base_r + step * CHUNK
        if writes[buf] is not None:
            writes[buf].wait()
        pltpu.async_copy(table_hbm.at[idx_v.at[step]],
                         rows_v.at[buf], gsems.at[buf]).wait()
        writes[buf] = pltpu.async_copy(
            rows_v.at[buf], out_hbm.at[pl.ds(row0, CHUNK)], wsems.at[buf])
    for w in writes:
        w.wait()


_gather_one = functools.partial(
    pl.kernel,
    out_type=jax.ShapeDtypeStruct((N_ROWS, EMBED_DIM), jnp.float32),
    mesh=plsc.VectorSubcoreMesh(core_axis_name="c", subcore_axis_name="s"),
    scratch_types=[
        pltpu.VMEM((N_CHUNKS, CHUNK), jnp.int32),
        pltpu.VMEM((NBUF, CHUNK, EMBED_DIM), jnp.float32),
        pltpu.SemaphoreType.DMA((NBUF,)),
        pltpu.SemaphoreType.DMA((NBUF,)),
    ],
    compiler_params=pltpu.CompilerParams(use_tc_tiling_on_sc=False),
)(_emb_kernel)


VB = 32768


def _tp_kernel(t_ref, o_ref):
    o_ref[:, 0:EMBED_DIM] = t_ref[...].T


def _transpose_pad(table_t):
    grid = (VOCAB + VB - 1) // VB
    return pl.pallas_call(
        _tp_kernel,
        grid=(grid,),
        in_specs=[pl.BlockSpec((EMBED_DIM, VB), lambda i: (0, i))],
        out_specs=pl.BlockSpec((VB, PAD_DIM), lambda i: (i, 0)),
        out_shape=jax.ShapeDtypeStruct((VOCAB, PAD_DIM), jnp.float32),
    )(table_t)


@jax.jit
def kernel(input, support, table):
    def prep_idx(ids):
        return (ids.astype(jnp.int32) * 2).reshape(N_ROWS // CHUNK, CHUNK)

    table_p = _transpose_pad(table.T)
    table_v = table_p.reshape(2 * VOCAB, EMBED_DIM)

    def to_native(o):
        t = o.reshape(BATCH, SEQ * EMBED_DIM).T
        return jnp.transpose(t.reshape(SEQ, EMBED_DIM, BATCH), (2, 0, 1))

    out_in = _gather_one(prep_idx(input), table_v)
    out_sup = _gather_one(prep_idx(support), table_v)
    return to_native(out_in), to_native(out_sup)


# speedup vs baseline: 1.0017x; 1.0017x over previous
import functools

import jax
import jax.numpy as jnp
from jax import lax
from jax.experimental import pallas as pl
from jax.experimental.pallas import tpu as pltpu
from jax.experimental.pallas import tpu_sc as plsc

VOCAB = 1000000
EMBED_DIM = 64
PAD_DIM = 128
BATCH = 4096
SEQ = 50

_INFO = plsc.get_sparse_core_info()
NC, NS = _INFO.num_cores, _INFO.num_subcores
NW = NC * NS

N_ROWS = BATCH * SEQ
PER_W = N_ROWS // NW
CHUNK = 800
N_CHUNKS = PER_W // CHUNK
NBUF = 2


def _emb_kernel(idx_hbm, table_hbm, out_hbm, idx_v, rows_v, gsems, wsems):
    wid = lax.axis_index("s") * NC + lax.axis_index("c")
    base_c = wid * N_CHUNKS
    base_r = wid * PER_W

    pltpu.sync_copy(idx_hbm.at[pl.ds(base_c, N_CHUNKS)], idx_v)

    writes = [None] * NBUF
    for step in range(N_CHUNKS):
        buf = step % NBUF
        row0 = base_r + step * CHUNK
        if writes[buf] is not None:
            writes[buf].wait()
        pltpu.async_copy(table_hbm.at[idx_v.at[step]],
                         rows_v.at[buf], gsems.at[buf]).wait()
        writes[buf] = pltpu.async_copy(
            rows_v.at[buf], out_hbm.at[pl.ds(row0, CHUNK)], wsems.at[buf])
    for w in writes:
        w.wait()


_gather_one = functools.partial(
    pl.kernel,
    out_type=jax.ShapeDtypeStruct((N_ROWS, EMBED_DIM), jnp.float32),
    mesh=plsc.VectorSubcoreMesh(core_axis_name="c", subcore_axis_name="s"),
    scratch_types=[
        pltpu.VMEM((N_CHUNKS, CHUNK), jnp.int32),
        pltpu.VMEM((NBUF, CHUNK, EMBED_DIM), jnp.float32),
        pltpu.SemaphoreType.DMA((NBUF,)),
        pltpu.SemaphoreType.DMA((NBUF,)),
    ],
    compiler_params=pltpu.CompilerParams(use_tc_tiling_on_sc=False),
)(_emb_kernel)


VB = 32768


def _tp_kernel(t_ref, o_ref):
    o_ref[:, 0:EMBED_DIM] = t_ref[...].T


def _transpose_pad(table_t):
    grid = (VOCAB + VB - 1) // VB
    return pl.pallas_call(
        _tp_kernel,
        grid=(grid,),
        in_specs=[pl.BlockSpec((EMBED_DIM, VB), lambda i: (0, i))],
        out_specs=pl.BlockSpec((VB, PAD_DIM), lambda i: (i, 0)),
        out_shape=jax.ShapeDtypeStruct((VOCAB, PAD_DIM), jnp.float32),
    )(table_t)


@jax.jit
def kernel(input, support, table):
    def prep_idx(ids):
        return (ids.astype(jnp.int32) * 2).reshape(N_ROWS // CHUNK, CHUNK)

    table_p = _transpose_pad(table.T)
    table_v = table_p.reshape(2 * VOCAB, EMBED_DIM)

    def to_native(o):
        t = o.reshape(BATCH, SEQ * EMBED_DIM).T
        return jnp.transpose(t.reshape(SEQ, EMBED_DIM, BATCH), (2, 0, 1))

    out_in = _gather_one(prep_idx(input), table_v)
    out_sup = _gather_one(prep_idx(support), table_v)
    return to_native(out_in), to_native(out_sup)
